# baseline (speedup 1.0000x reference)
import functools

import jax
import jax.numpy as jnp
from jax import lax
from jax.experimental import pallas as pl
from jax.experimental.pallas import tpu as pltpu, tpu_sc as plsc

_INFO = plsc.get_sparse_core_info()
_NW = _INFO.num_cores * _INFO.num_subcores

_N = 1048576
_NB = _N // 128
_BPW = _NB // _NW
_CB = 32
_CHUNKS = _BPW // _CB
_S = _CB * 512


def _body(in_hbm, out_hbm, in_v, out_v):
    wid = lax.axis_index("s") * _INFO.num_cores + lax.axis_index("c")
    zeros = jnp.zeros((16,), jnp.float32)
    base_blk = wid * _BPW

    def chunk(ch, carry):
        off = (base_blk + ch * _CB) * 512
        pltpu.sync_copy(in_hbm.at[pl.ds(off, _S)], in_v)

        def blk(b, c2):
            ib = b * 512
            for j in range(8):
                s = ib + j * 16
                d0 = in_v[pl.ds(s, 16)]
                d1 = in_v[pl.ds(s + 128, 16)]
                d2 = in_v[pl.ds(s + 256, 16)]
                out_v[pl.ds(s, 16)] = zeros
                out_v[pl.ds(s + 128, 16)] = -d2
                out_v[pl.ds(s + 256, 16)] = d1
                out_v[pl.ds(_S + s, 16)] = d2
                out_v[pl.ds(_S + s + 128, 16)] = zeros
                out_v[pl.ds(_S + s + 256, 16)] = -d0
                out_v[pl.ds(2 * _S + s, 16)] = -d1
                out_v[pl.ds(2 * _S + s + 128, 16)] = d0
                out_v[pl.ds(2 * _S + s + 256, 16)] = zeros
            return c2

        lax.fori_loop(0, _CB, blk, 0)
        for r in range(3):
            pltpu.sync_copy(
                out_v.at[pl.ds(r * _S, _S)],
                out_hbm.at[pl.ds(r * 4 * _N + off, _S)],
            )
        return carry

    lax.fori_loop(0, _CHUNKS, chunk, 0)


_skew = functools.partial(
    pl.kernel,
    out_type=jax.ShapeDtypeStruct((12 * _N,), jnp.float32),
    mesh=plsc.VectorSubcoreMesh(core_axis_name="c", subcore_axis_name="s"),
    scratch_types=[
        pltpu.VMEM((_S,), jnp.float32),
        pltpu.VMEM((3 * _S,), jnp.float32),
    ],
    compiler_params=pltpu.CompilerParams(needs_layout_passes=False),
)(_body)


def kernel(dw):
    n = dw.shape[0]
    nb = n // 128
    dw4 = jnp.pad(dw, ((0, 0), (0, 1)))
    x = dw4.T.reshape(4, nb, 128).transpose(1, 0, 2).reshape(4 * n)
    o = _skew(x)
    o4 = o.reshape(3, nb, 4, 128).transpose(1, 3, 0, 2)
    return o4[:, :, :, :3].reshape(n, 3, 3)


# speedup vs baseline: 3.9559x; 3.9559x over previous
import jax
import jax.numpy as jnp
from jax.experimental import pallas as pl

_B_MAX = 262144


def _body(x_ref, o_ref):
    x = x_ref[...]
    z = jnp.zeros_like(x[0:1])
    d0, d1, d2 = x[0:1], x[1:2], x[2:3]
    o_ref[0] = jnp.concatenate([z, -d2, d1], axis=0)
    o_ref[1] = jnp.concatenate([d2, z, -d0], axis=0)
    o_ref[2] = jnp.concatenate([-d1, d0, z], axis=0)


def kernel(dw):
    n = dw.shape[0]
    b = n if n <= _B_MAX else _B_MAX
    call = pl.pallas_call(
        _body,
        out_shape=jax.ShapeDtypeStruct((3, 3, n), jnp.float32),
        grid=(n // b,),
        in_specs=[pl.BlockSpec((3, b), lambda i: (0, i))],
        out_specs=pl.BlockSpec((3, 3, b), lambda i: (0, 0, i)),
    )
    o = call(dw.T)
    return o.transpose(2, 0, 1)
